# baseline (speedup 1.0000x reference)
import functools

import jax
import jax.numpy as jnp
from jax.experimental import pallas as pl
from jax.experimental.pallas import tpu as pltpu

_N = 50000
_A = 512
_CHUNK = 1000
_NBUF = 8
_NSTEPS = _N // _CHUNK


def _pipelined_kernel(x_hbm, w_ref, b_ref, o_hbm, xbuf, obuf, in_sems, out_sems):
    w = w_ref[...]
    bias = b_ref[...]

    def in_copy(i, s):
        return pltpu.make_async_copy(
            x_hbm.at[pl.ds(i * _CHUNK, _CHUNK), :], xbuf.at[s], in_sems.at[s])

    def out_copy(i, s):
        return pltpu.make_async_copy(
            obuf.at[s], o_hbm.at[pl.ds(i * _CHUNK, _CHUNK), :], out_sems.at[s])

    for s in range(_NBUF):
        in_copy(s, s).start()

    for i in range(_NSTEPS):
        s = i % _NBUF
        in_copy(i, s).wait()
        if i >= _NBUF:
            out_copy(i - _NBUF, s).wait()
        xb = xbuf[s]
        acc = jax.lax.dot_general(
            xb, w,
            dimension_numbers=(((1,), (1,)), ((), ())),
            preferred_element_type=jnp.float32,
        )
        obuf[s] = xb + 0.1 * acc + 0.1 * bias
        out_copy(i, s).start()
        if i + _NBUF < _NSTEPS:
            in_copy(i + _NBUF, s).start()

    for i in range(_NSTEPS - _NBUF, _NSTEPS):
        out_copy(i, i % _NBUF).wait()


@functools.partial(jax.jit, static_argnames=())
def kernel(x, W, b):
    b2 = b.reshape(1, _A)
    return pl.pallas_call(
        _pipelined_kernel,
        in_specs=[
            pl.BlockSpec(memory_space=pltpu.HBM),
            pl.BlockSpec((_A, _A), lambda: (0, 0)),
            pl.BlockSpec((1, _A), lambda: (0, 0)),
        ],
        out_specs=pl.BlockSpec(memory_space=pltpu.HBM),
        out_shape=jax.ShapeDtypeStruct((_N, _A), jnp.float32),
        scratch_shapes=[
            pltpu.VMEM((_NBUF, _CHUNK, _A), jnp.float32),
            pltpu.VMEM((_NBUF, _CHUNK, _A), jnp.float32),
            pltpu.SemaphoreType.DMA((_NBUF,)),
            pltpu.SemaphoreType.DMA((_NBUF,)),
        ],
    )(x, W, b2)


# speedup vs baseline: 1.0011x; 1.0011x over previous
import functools

import jax
import jax.numpy as jnp
from jax.experimental import pallas as pl
from jax.experimental.pallas import tpu as pltpu

_N = 50000
_A = 512
_CHUNK = 1000
_NBUF = 10
_NSTEPS = _N // _CHUNK


def _pipelined_kernel(x_hbm, w_ref, b_ref, o_hbm, xbuf, obuf, in_sems, out_sems):
    w = w_ref[...]
    bias = b_ref[...]

    def in_copy(i, s):
        return pltpu.make_async_copy(
            x_hbm.at[pl.ds(i * _CHUNK, _CHUNK), :], xbuf.at[s], in_sems.at[s])

    def out_copy(i, s):
        return pltpu.make_async_copy(
            obuf.at[s], o_hbm.at[pl.ds(i * _CHUNK, _CHUNK), :], out_sems.at[s])

    for s in range(_NBUF):
        in_copy(s, s).start()

    for i in range(_NSTEPS):
        s = i % _NBUF
        in_copy(i, s).wait()
        if i >= _NBUF:
            out_copy(i - _NBUF, s).wait()
        xb = xbuf[s]
        acc = jax.lax.dot_general(
            xb, w,
            dimension_numbers=(((1,), (1,)), ((), ())),
            preferred_element_type=jnp.float32,
        )
        obuf[s] = xb + 0.1 * acc + 0.1 * bias
        out_copy(i, s).start()
        if i + _NBUF < _NSTEPS:
            in_copy(i + _NBUF, s).start()

    for i in range(_NSTEPS - _NBUF, _NSTEPS):
        out_copy(i, i % _NBUF).wait()


@functools.partial(jax.jit, static_argnames=())
def kernel(x, W, b):
    b2 = b.reshape(1, _A)
    return pl.pallas_call(
        _pipelined_kernel,
        in_specs=[
            pl.BlockSpec(memory_space=pltpu.HBM),
            pl.BlockSpec((_A, _A), lambda: (0, 0)),
            pl.BlockSpec((1, _A), lambda: (0, 0)),
        ],
        out_specs=pl.BlockSpec(memory_space=pltpu.HBM),
        out_shape=jax.ShapeDtypeStruct((_N, _A), jnp.float32),
        scratch_shapes=[
            pltpu.VMEM((_NBUF, _CHUNK, _A), jnp.float32),
            pltpu.VMEM((_NBUF, _CHUNK, _A), jnp.float32),
            pltpu.SemaphoreType.DMA((_NBUF,)),
            pltpu.SemaphoreType.DMA((_NBUF,)),
        ],
    )(x, W, b2)


# speedup vs baseline: 1.0070x; 1.0059x over previous
import functools

import jax
import jax.numpy as jnp
from jax.experimental import pallas as pl
from jax.experimental.pallas import tpu as pltpu

_N = 50000
_A = 512
_CHUNK = 2000
_NBUF = 6
_NSTEPS = _N // _CHUNK


def _pipelined_kernel(x_hbm, w_ref, b_ref, o_hbm, xbuf, obuf, in_sems, out_sems):
    w = w_ref[...]
    bias = b_ref[...]

    def in_copy(i, s):
        return pltpu.make_async_copy(
            x_hbm.at[pl.ds(i * _CHUNK, _CHUNK), :], xbuf.at[s], in_sems.at[s])

    def out_copy(i, s):
        return pltpu.make_async_copy(
            obuf.at[s], o_hbm.at[pl.ds(i * _CHUNK, _CHUNK), :], out_sems.at[s])

    for s in range(_NBUF):
        in_copy(s, s).start()

    for i in range(_NSTEPS):
        s = i % _NBUF
        in_copy(i, s).wait()
        if i >= _NBUF:
            out_copy(i - _NBUF, s).wait()
        xb = xbuf[s]
        acc = jax.lax.dot_general(
            xb, w,
            dimension_numbers=(((1,), (1,)), ((), ())),
            preferred_element_type=jnp.float32,
        )
        obuf[s] = xb + 0.1 * acc + 0.1 * bias
        out_copy(i, s).start()
        if i + _NBUF < _NSTEPS:
            in_copy(i + _NBUF, s).start()

    for i in range(_NSTEPS - _NBUF, _NSTEPS):
        out_copy(i, i % _NBUF).wait()


@functools.partial(jax.jit, static_argnames=())
def kernel(x, W, b):
    b2 = b.reshape(1, _A)
    return pl.pallas_call(
        _pipelined_kernel,
        in_specs=[
            pl.BlockSpec(memory_space=pltpu.HBM),
            pl.BlockSpec((_A, _A), lambda: (0, 0)),
            pl.BlockSpec((1, _A), lambda: (0, 0)),
        ],
        out_specs=pl.BlockSpec(memory_space=pltpu.HBM),
        out_shape=jax.ShapeDtypeStruct((_N, _A), jnp.float32),
        scratch_shapes=[
            pltpu.VMEM((_NBUF, _CHUNK, _A), jnp.float32),
            pltpu.VMEM((_NBUF, _CHUNK, _A), jnp.float32),
            pltpu.SemaphoreType.DMA((_NBUF,)),
            pltpu.SemaphoreType.DMA((_NBUF,)),
        ],
    )(x, W, b2)
